# baseline (speedup 1.0000x reference)
import jax
import jax.numpy as jnp
from jax.experimental import pallas as pl


def _mm_kernel(a_ref, w_ref, b_ref, o_ref):
    o_ref[...] = jnp.dot(a_ref[...], w_ref[...], preferred_element_type=jnp.float32) + b_ref[...]


def kernel(x, edge_index, batch, W1, b1, W2, b2, W3, b3, Wc, bc):
    n = x.shape[0]
    loop = jnp.arange(n, dtype=edge_index.dtype)
    src = jnp.concatenate([edge_index[0], loop])
    dst = jnp.concatenate([edge_index[1], loop])
    deg = jax.ops.segment_sum(jnp.ones_like(dst, dtype=x.dtype), dst, num_segments=n)
    dis = jnp.where(deg > 0, 1.0 / jnp.sqrt(deg), 0.0)
    norm = dis[src] * dis[dst]

    def conv(h, W, b):
        hw = h @ W
        msg = hw[src] * norm[:, None]
        return jax.ops.segment_sum(msg, dst, num_segments=n) + b

    h = jax.nn.relu(conv(x, W1, b1))
    h = jax.nn.relu(conv(h, W2, b2))
    h = conv(h, W3, b3)
    G = 64
    sums = jax.ops.segment_sum(h, batch, num_segments=G)
    cnts = jax.ops.segment_sum(jnp.ones((n,), h.dtype), batch, num_segments=G)
    pooled = sums / jnp.maximum(cnts, 1.0)[:, None]
    logits = pl.pallas_call(
        _mm_kernel,
        out_shape=jax.ShapeDtypeStruct((G, bc.shape[0]), jnp.float32),
    )(pooled, Wc, bc[None, :])
    return logits


# speedup vs baseline: 10.2599x; 10.2599x over previous
import functools

import jax
import jax.numpy as jnp
from jax import lax
from jax.experimental import pallas as pl
from jax.experimental.pallas import tpu as pltpu
from jax.experimental.pallas import tpu_sc as plsc

N = 10000
H = 256
G = 64
N_PAD = 10112
RPT = 632
NB = 158
NB_ES = 79
E_PAD = 16 * NB * 128

_MESH = dict(core_axis_name="c", subcore_axis_name="s",
             num_cores=2, num_subcores=16)


@functools.partial(
    pl.kernel,
    out_type=jax.ShapeDtypeStruct((2 * N_PAD, 128), jnp.float32),
    mesh=plsc.VectorSubcoreMesh(**_MESH),
    scratch_types=[
        pltpu.VMEM((NB_ES, 128), jnp.int32),
        pltpu.VMEM((128, 128), jnp.float32),
        pltpu.VMEM_SHARED((N_PAD, 128), jnp.float32),
    ],
)
def _deg(ones_hbm, dst_hbm, z_hbm, out_hbm, dst_v, buf, acc):
    c = lax.axis_index("c")
    s = lax.axis_index("s")
    pltpu.sync_copy(dst_hbm.at[c, s], dst_v)
    pltpu.sync_copy(ones_hbm, buf)
    r0 = s * RPT
    pltpu.sync_copy(z_hbm, acc.at[pl.ds(r0, RPT)])
    plsc.subcore_barrier()

    def body(b, carry):
        pltpu.sync_copy(buf, acc.at[dst_v.at[b]], add=True)
        return carry

    lax.fori_loop(0, NB_ES, body, 0)
    plsc.subcore_barrier()
    pltpu.sync_copy(acc.at[pl.ds(r0, RPT)],
                    out_hbm.at[pl.ds(c * N_PAD + r0, RPT)])


@functools.partial(
    pl.kernel,
    out_type=jax.ShapeDtypeStruct((2 * N_PAD, 128), jnp.float32),
    mesh=plsc.VectorSubcoreMesh(**_MESH),
    scratch_types=[
        pltpu.VMEM((NB_ES, 128), jnp.int32),
        pltpu.VMEM((NB_ES, 128), jnp.int32),
        pltpu.VMEM((128, 128), jnp.float32),
        pltpu.VMEM_SHARED((N_PAD, 128), jnp.float32),
        pltpu.SemaphoreType.DMA,
    ],
)
def _spmm_es(g_hbm, src_hbm, dst_hbm, z_hbm, out_hbm,
             src_v, dst_v, buf, acc, sem):
    c = lax.axis_index("c")
    s = lax.axis_index("s")
    pltpu.sync_copy(src_hbm.at[c, s], src_v)
    pltpu.sync_copy(dst_hbm.at[c, s], dst_v)
    r0 = s * RPT

    @pl.when(c == 0)
    def _():
        pltpu.sync_copy(g_hbm.at[pl.ds(r0, RPT)], acc.at[pl.ds(r0, RPT)])

    @pl.when(c == 1)
    def _():
        pltpu.sync_copy(z_hbm, acc.at[pl.ds(r0, RPT)])

    plsc.subcore_barrier()

    def body(b, carry):
        pltpu.async_copy(g_hbm.at[src_v.at[b]], buf, sem).wait()
        pltpu.sync_copy(buf, acc.at[dst_v.at[b]], add=True)
        return carry

    lax.fori_loop(0, NB_ES, body, 0)
    plsc.subcore_barrier()
    pltpu.sync_copy(acc.at[pl.ds(r0, RPT)],
                    out_hbm.at[pl.ds(c * N_PAD + r0, RPT)])


@functools.partial(
    pl.kernel,
    out_type=jax.ShapeDtypeStruct((2 * N_PAD, 128), jnp.float32),
    mesh=plsc.VectorSubcoreMesh(**_MESH),
    scratch_types=[
        pltpu.VMEM((NB_ES, 128), jnp.int32),
        pltpu.VMEM((NB_ES, 128), jnp.int32),
        pltpu.VMEM((128, 128), jnp.float32),
        pltpu.VMEM_SHARED((N_PAD, 128), jnp.float32),
        pltpu.SemaphoreType.DMA,
    ],
)
def _spmm_fs(g_hbm, src_hbm, dst_hbm, out_hbm, src_v, dst_v, buf, acc, sem):
    c = lax.axis_index("c")
    s = lax.axis_index("s")
    r0 = s * RPT
    pltpu.sync_copy(g_hbm.at[pl.ds(c * N_PAD + r0, RPT)],
                    acc.at[pl.ds(r0, RPT)])
    plsc.subcore_barrier()

    def chunk(k, carry):
        pltpu.sync_copy(src_hbm.at[c, s, k], src_v)
        pltpu.sync_copy(dst_hbm.at[s, k], dst_v)

        def body(b, carry2):
            pltpu.async_copy(g_hbm.at[src_v.at[b]], buf, sem).wait()
            pltpu.sync_copy(buf, acc.at[dst_v.at[b]], add=True)
            return carry2

        lax.fori_loop(0, NB_ES, body, 0)
        return carry

    lax.fori_loop(0, 2, chunk, 0)
    plsc.subcore_barrier()
    pltpu.sync_copy(acc.at[pl.ds(r0, RPT)],
                    out_hbm.at[pl.ds(c * N_PAD + r0, RPT)])


def _scale_kernel(d0_ref, d1_ref, x_ref, g_ref, dis_ref):
    deg = d0_ref[:, 0:1] + d1_ref[:, 0:1] + 1.0
    dis = lax.rsqrt(deg)
    g_ref[...] = dis * x_ref[...]
    dis_ref[...] = jnp.broadcast_to(dis, (RPT, 8))


def _tc_scale(degp, x_pad):
    return pl.pallas_call(
        _scale_kernel,
        grid=(16,),
        in_specs=[
            pl.BlockSpec((RPT, 128), lambda i: (i, 0)),
            pl.BlockSpec((RPT, 128), lambda i: (16 + i, 0)),
            pl.BlockSpec((RPT, 128), lambda i: (i, 0)),
        ],
        out_specs=[
            pl.BlockSpec((RPT, 128), lambda i: (i, 0)),
            pl.BlockSpec((RPT, 8), lambda i: (i, 0)),
        ],
        out_shape=[
            jax.ShapeDtypeStruct((N_PAD, 128), jnp.float32),
            jax.ShapeDtypeStruct((N_PAD, 8), jnp.float32),
        ],
    )(degp, degp, x_pad)


def _layer_es_kernel(dis_ref, s0_ref, s1_ref, w_ref, b_ref, o_ref):
    c = pl.program_id(0) // 16
    dis = dis_ref[:, 0:1]
    h = dis * (s0_ref[...] + s1_ref[...])
    h = jnp.dot(h, w_ref[...], preferred_element_type=jnp.float32) + b_ref[...]
    h = jnp.maximum(h, 0.0)
    g = dis * h
    o_ref[...] = jnp.where(c == 0, g[:, :128], g[:, 128:])


def _layer_fs_kernel(dis_ref, s0_ref, s1_ref, w_ref, b_ref, o_ref):
    c = pl.program_id(0) // 16
    dis = dis_ref[:, 0:1]
    h = dis * jnp.concatenate([s0_ref[...], s1_ref[...]], axis=1)
    h = jnp.dot(h, w_ref[...], preferred_element_type=jnp.float32) + b_ref[...]
    h = jnp.maximum(h, 0.0)
    g = dis * h
    o_ref[...] = jnp.where(c == 0, g[:, :128], g[:, 128:])


def _tc_layer(body_fn, dis8, s_flat, W, b, dh_in):
    return pl.pallas_call(
        body_fn,
        grid=(32,),
        in_specs=[
            pl.BlockSpec((RPT, 8), lambda i: (i % 16, 0)),
            pl.BlockSpec((RPT, dh_in), lambda i: (i % 16, 0)),
            pl.BlockSpec((RPT, dh_in), lambda i: (16 + i % 16, 0)),
            pl.BlockSpec(W.shape, lambda i: (0, 0)),
            pl.BlockSpec((1, H), lambda i: (0, 0)),
        ],
        out_specs=pl.BlockSpec((RPT, 128), lambda i: (i, 0)),
        out_shape=jax.ShapeDtypeStruct((2 * N_PAD, 128), jnp.float32),
    )(dis8, s_flat, s_flat, W, b)


def _final_kernel(dis_ref, s0_ref, s1_ref, w_ref, b_ref, batch_ref,
                  wc_ref, bc_ref, o_ref, sums_ref, cnts_ref):
    i = pl.program_id(0)
    dis = dis_ref[:, 0:1]
    h = dis * jnp.concatenate([s0_ref[...], s1_ref[...]], axis=1)
    h = jnp.dot(h, w_ref[...], preferred_element_type=jnp.float32) + b_ref[...]

    gids = lax.broadcasted_iota(jnp.int32, (G, RPT), 0)
    onehot = (gids == batch_ref[0]).astype(jnp.float32)

    @pl.when(i == 0)
    def _():
        sums_ref[...] = jnp.zeros_like(sums_ref)
        cnts_ref[...] = jnp.zeros_like(cnts_ref)

    sums_ref[...] += jnp.dot(onehot, h, preferred_element_type=jnp.float32)
    cnts_ref[...] += jnp.sum(onehot, axis=1, keepdims=True)

    @pl.when(i == 15)
    def _():
        pooled = sums_ref[...] / jnp.maximum(cnts_ref[...], 1.0)
        o_ref[...] = (jnp.dot(pooled, wc_ref[...],
                              preferred_element_type=jnp.float32)
                      + bc_ref[...])


def _tc_final(dis8, s_flat, W3, b3, batch3, Wc, bc):
    return pl.pallas_call(
        _final_kernel,
        grid=(16,),
        in_specs=[
            pl.BlockSpec((RPT, 8), lambda i: (i, 0)),
            pl.BlockSpec((RPT, 128), lambda i: (i, 0)),
            pl.BlockSpec((RPT, 128), lambda i: (16 + i, 0)),
            pl.BlockSpec((H, H), lambda i: (0, 0)),
            pl.BlockSpec((1, H), lambda i: (0, 0)),
            pl.BlockSpec((1, 1, RPT), lambda i: (i, 0, 0)),
            pl.BlockSpec((H, 10), lambda i: (0, 0)),
            pl.BlockSpec((1, 10), lambda i: (0, 0)),
        ],
        out_specs=pl.BlockSpec((G, 10), lambda i: (0, 0)),
        out_shape=jax.ShapeDtypeStruct((G, 10), jnp.float32),
        scratch_shapes=[
            pltpu.VMEM((G, H), jnp.float32),
            pltpu.VMEM((G, 1), jnp.float32),
        ],
    )(dis8, s_flat, s_flat, W3, b3, batch3, Wc, bc)


def kernel(x, edge_index, batch, W1, b1, W2, b2, W3, b3, Wc, bc):
    src = edge_index[0].astype(jnp.int32)
    dst = edge_index[1].astype(jnp.int32)
    src = jnp.pad(src, (0, E_PAD - src.shape[0]))
    dst = jnp.pad(dst, (0, E_PAD - dst.shape[0]), constant_values=N)
    src_es = src.reshape(2, 16, NB_ES, 128)
    dst_es = dst.reshape(2, 16, NB_ES, 128)
    src_fs = jnp.stack([src, src + N_PAD]).reshape(2, 16, 2, NB_ES, 128)
    dst_fs = dst.reshape(16, 2, NB_ES, 128)

    x_pad = jnp.pad(x, ((0, N_PAD - N), (0, 0)))
    batch3 = jnp.pad(batch.astype(jnp.int32), (0, N_PAD - N),
                     constant_values=G).reshape(16, 1, RPT)
    ones128 = jnp.ones((128, 128), jnp.float32)
    zeros_rp = jnp.zeros((RPT, 128), jnp.float32)

    degp = _deg(ones128, dst_es, zeros_rp)
    g1, dis8 = _tc_scale(degp, x_pad)
    s1 = _spmm_es(g1, src_es, dst_es, zeros_rp)
    g2 = _tc_layer(_layer_es_kernel, dis8, s1, W1, b1[None, :], 128)
    s2 = _spmm_fs(g2, src_fs, dst_fs)
    g3 = _tc_layer(_layer_fs_kernel, dis8, s2, W2, b2[None, :], 128)
    s3 = _spmm_fs(g3, src_fs, dst_fs)
    return _tc_final(dis8, s3, W3, b3[None, :], batch3, Wc, bc[None, :])
